# Initial kernel scaffold; baseline (speedup 1.0000x reference)
import jax
import jax.numpy as jnp
from jax.experimental import pallas as pl


def kernel(src, E0, E1, E2, P0, P1, P2):
    raise NotImplementedError("write your pallas kernel here")


# speedup vs baseline: 3.6322x; 3.6322x over previous
import functools

import jax
import jax.numpy as jnp
from jax import lax
from jax.experimental import pallas as pl
from jax.experimental.pallas import tpu as pltpu
from jax.experimental.pallas import tpu_sc as plsc

B, S, D, V = 1024, 200, 128, 100000
NC, NS = 2, 16
NW = NC * NS
NPW = (B * S) // NW
C = 40
CHUNKS = NPW // C
PC = S // C
LANES = 16
CD = D // LANES


def _matryoshka_sc(srcf, E0, E1, E2, P0f, P1f, P2f):
    mesh = plsc.VectorSubcoreMesh(core_axis_name="c", subcore_axis_name="s")

    @functools.partial(
        pl.kernel,
        mesh=mesh,
        out_type=jax.ShapeDtypeStruct((B * S, D), jnp.float32),
        scratch_types=[
            pltpu.VMEM((C,), jnp.int32),
            pltpu.VMEM((C, D), jnp.float32),
            pltpu.VMEM((C, D), jnp.float32),
            pltpu.VMEM((C, D), jnp.float32),
            pltpu.VMEM((C, D), jnp.float32),
            pltpu.VMEM((S, D), jnp.float32),
            pltpu.VMEM((S, D), jnp.float32),
            pltpu.SemaphoreType.DMA,
            pltpu.SemaphoreType.DMA,
            pltpu.SemaphoreType.DMA,
        ],
    )
    def k(src_hbm, e0, e1, e2, p0, p1, p2, out_hbm,
          idx_v, g0, g1, g2, ob, ppos, ptmp, sem0, sem1, sem2):
        wid = lax.axis_index("s") * NC + lax.axis_index("c")
        wbase = wid * NPW

        pltpu.sync_copy(p0.at[pl.ds(0, S)], ppos)
        pltpu.sync_copy(p1.at[pl.ds(0, S)], ptmp)

        def _acc_row(r, carry):
            for c in range(CD):
                sl = pl.ds(c * LANES, LANES)
                ppos[r, sl] = ppos[r, sl] + ptmp[r, sl]
            return carry

        lax.fori_loop(0, S, _acc_row, 0)
        pltpu.sync_copy(p2.at[pl.ds(0, S)], ptmp)
        lax.fori_loop(0, S, _acc_row, 0)

        def chunk_body(kk, carry):
            base = wbase + kk * C
            pbase = (kk % PC) * C
            pltpu.sync_copy(src_hbm.at[pl.ds(base, C)], idx_v)
            cp0 = pltpu.async_copy(e0.at[idx_v], g0, sem0)
            cp1 = pltpu.async_copy(e1.at[idx_v], g1, sem1)
            cp2 = pltpu.async_copy(e2.at[idx_v], g2, sem2)
            cp0.wait()
            cp1.wait()
            cp2.wait()

            def row_body(r, rc):
                pr = pbase + r
                for c in range(CD):
                    sl = pl.ds(c * LANES, LANES)
                    ob[r, sl] = (g0[r, sl] + g1[r, sl] + g2[r, sl]
                                 + ppos[pr, sl])
                return rc

            lax.fori_loop(0, C, row_body, 0)
            pltpu.sync_copy(ob, out_hbm.at[pl.ds(base, C)])
            return carry

        lax.fori_loop(0, CHUNKS, chunk_body, 0)

    return k(srcf, E0, E1, E2, P0f, P1f, P2f)


def kernel(src, E0, E1, E2, P0, P1, P2):
    srcf = src.reshape(B * S)
    P0f = P0.reshape(-1, D)
    P1f = P1.reshape(-1, D)
    P2f = P2.reshape(-1, D)
    out = _matryoshka_sc(srcf, E0, E1, E2, P0f, P1f, P2f)
    return out.reshape(B, S, D)


# speedup vs baseline: 7.1257x; 1.9618x over previous
import functools

import jax
import jax.numpy as jnp
from jax import lax
from jax.experimental import pallas as pl
from jax.experimental.pallas import tpu as pltpu
from jax.experimental.pallas import tpu_sc as plsc

B, S, D, V = 1024, 200, 128, 100000
NC, NS = 2, 16
NW = NC * NS
NPW = (B * S) // NW
C = 40
CHUNKS = NPW // C
PC = S // C
LANES = 16
CD = D // LANES
NBUF = 2


def _matryoshka_sc(src3, E0, E1, E2, P0f, P1f, P2f):
    mesh = plsc.VectorSubcoreMesh(core_axis_name="c", subcore_axis_name="s")

    @functools.partial(
        pl.kernel,
        mesh=mesh,
        out_type=jax.ShapeDtypeStruct((B * S, D), jnp.float32),
        scratch_types=[
            pltpu.VMEM((CHUNKS, C), jnp.int32),
            pltpu.VMEM((NBUF, C, D), jnp.float32),
            pltpu.VMEM((NBUF, C, D), jnp.float32),
            pltpu.VMEM((NBUF, C, D), jnp.float32),
            pltpu.VMEM((NBUF, C, D), jnp.float32),
            pltpu.VMEM((S, D), jnp.float32),
            pltpu.VMEM((S, D), jnp.float32),
            pltpu.SemaphoreType.DMA,
            pltpu.SemaphoreType.DMA,
            pltpu.SemaphoreType.DMA,
            pltpu.SemaphoreType.DMA,
        ],
    )
    def k(src_hbm, e0, e1, e2, p0, p1, p2, out_hbm,
          idx_all, g0, g1, g2, ob, ppos, ptmp,
          semg0, semg1, semo0, semo1):
        semg = (semg0, semg1)
        semo = (semo0, semo1)
        wid = lax.axis_index("s") * NC + lax.axis_index("c")
        wbase = wid * NPW

        pltpu.sync_copy(src_hbm.at[wid], idx_all)

        def issue(ck, b):
            idx = idx_all.at[ck]
            pltpu.async_copy(e0.at[idx], g0.at[b], semg[b])
            pltpu.async_copy(e1.at[idx], g1.at[b], semg[b])
            pltpu.async_copy(e2.at[idx], g2.at[b], semg[b])

        def wait_gathers(b):
            for gbuf in (g0, g1, g2):
                pltpu.make_async_copy(
                    e0.at[pl.ds(0, C)], gbuf.at[b], semg[b]).wait()

        issue(0, 0)
        issue(1, 1)

        pltpu.sync_copy(p0.at[pl.ds(0, S)], ppos)
        pltpu.sync_copy(p1.at[pl.ds(0, S)], ptmp)

        def _acc_row(r, carry):
            for c in range(CD):
                sl = pl.ds(c * LANES, LANES)
                ppos[r, sl] = ppos[r, sl] + ptmp[r, sl]
            return carry

        lax.fori_loop(0, S, _acc_row, 0)
        pltpu.sync_copy(p2.at[pl.ds(0, S)], ptmp)
        lax.fori_loop(0, S, _acc_row, 0)

        def body(i, carry):
            for b in range(NBUF):
                ck = i * NBUF + b
                wait_gathers(b)

                @pl.when(i > 0)
                def _():
                    pltpu.make_async_copy(
                        ob.at[b], out_hbm.at[pl.ds(0, C)], semo[b]).wait()

                pbase = (ck % PC) * C

                def row_body(r, rc):
                    pr = pbase + r
                    for c in range(CD):
                        sl = pl.ds(c * LANES, LANES)
                        ob[b, r, sl] = (g0[b, r, sl] + g1[b, r, sl]
                                        + g2[b, r, sl] + ppos[pr, sl])
                    return rc

                lax.fori_loop(0, C, row_body, 0)
                pltpu.async_copy(
                    ob.at[b], out_hbm.at[pl.ds(wbase + ck * C, C)], semo[b])

                nk = ck + NBUF

                @pl.when(nk < CHUNKS)
                def _():
                    issue(nk, b)
            return carry

        lax.fori_loop(0, CHUNKS // NBUF, body, 0)
        for b in range(NBUF):
            pltpu.make_async_copy(
                ob.at[b], out_hbm.at[pl.ds(0, C)], semo[b]).wait()

    return k(src3, E0, E1, E2, P0f, P1f, P2f)


def kernel(src, E0, E1, E2, P0, P1, P2):
    src3 = src.reshape(NW, CHUNKS, C)
    P0f = P0.reshape(-1, D)
    P1f = P1.reshape(-1, D)
    P2f = P2.reshape(-1, D)
    out = _matryoshka_sc(src3, E0, E1, E2, P0f, P1f, P2f)
    return out.reshape(B, S, D)


# speedup vs baseline: 9.8019x; 1.3756x over previous
import functools

import jax
import jax.numpy as jnp
from jax import lax
from jax.experimental import pallas as pl
from jax.experimental.pallas import tpu as pltpu
from jax.experimental.pallas import tpu_sc as plsc

B, S, D, V = 1024, 200, 128, 100000
NC, NS = 2, 16
NW = NC * NS
NPW = (B * S) // NW
C = 40
CHUNKS = NPW // C
PC = S // C
LANES = 16
CD = D // LANES
NBUF = 2


def _matryoshka_sc(src3, E0, E1, E2, P0f, P1f, P2f):
    mesh = plsc.VectorSubcoreMesh(core_axis_name="c", subcore_axis_name="s")

    @functools.partial(
        pl.kernel,
        mesh=mesh,
        out_type=jax.ShapeDtypeStruct((B * S, D), jnp.float32),
        scratch_types=[
            pltpu.VMEM((CHUNKS, C), jnp.int32),
            pltpu.VMEM((NBUF, C, D), jnp.float32),
            pltpu.VMEM((NBUF, C, D), jnp.float32),
            pltpu.VMEM((NBUF, C, D), jnp.float32),
            pltpu.VMEM((NBUF, C, D), jnp.float32),
            pltpu.VMEM((S, D), jnp.float32),
            pltpu.VMEM((S, D), jnp.float32),
            pltpu.SemaphoreType.DMA,
            pltpu.SemaphoreType.DMA,
            pltpu.SemaphoreType.DMA,
            pltpu.SemaphoreType.DMA,
        ],
    )
    def k(src_hbm, e0, e1, e2, p0, p1, p2, out_hbm,
          idx_all, g0, g1, g2, ob, ppos, ptmp,
          semg0, semg1, semo0, semo1):
        semg = (semg0, semg1)
        semo = (semo0, semo1)
        wid = lax.axis_index("s") * NC + lax.axis_index("c")
        wbase = wid * NPW

        pltpu.sync_copy(src_hbm.at[wid], idx_all)

        def issue(ck, b):
            idx = idx_all.at[ck]
            pltpu.async_copy(e0.at[idx], g0.at[b], semg[b])
            pltpu.async_copy(e1.at[idx], g1.at[b], semg[b])
            pltpu.async_copy(e2.at[idx], g2.at[b], semg[b])

        def wait_gathers(b):
            for gbuf in (g0, g1, g2):
                pltpu.make_async_copy(
                    e0.at[pl.ds(0, C)], gbuf.at[b], semg[b]).wait()

        issue(0, 0)
        issue(1, 1)

        pltpu.sync_copy(p0.at[pl.ds(0, S)], ppos)
        pltpu.sync_copy(p1.at[pl.ds(0, S)], ptmp)

        def _acc_row(r):
            for c in range(CD):
                sl = pl.ds(c * LANES, LANES)
                ppos[r, sl] = ppos[r, sl] + ptmp[r, sl]

        plsc.parallel_loop(0, S, 1, unroll=2)(_acc_row)
        pltpu.sync_copy(p2.at[pl.ds(0, S)], ptmp)
        plsc.parallel_loop(0, S, 1, unroll=2)(_acc_row)

        def body(i, carry):
            for b in range(NBUF):
                ck = i * NBUF + b
                wait_gathers(b)

                @pl.when(i > 0)
                def _():
                    pltpu.make_async_copy(
                        ob.at[b], out_hbm.at[pl.ds(0, C)], semo[b]).wait()

                pbase = (ck % PC) * C

                def row_body(r):
                    pr = pbase + r
                    for c in range(CD):
                        sl = pl.ds(c * LANES, LANES)
                        ob[b, r, sl] = (g0[b, r, sl] + g1[b, r, sl]
                                        + g2[b, r, sl] + ppos[pr, sl])

                plsc.parallel_loop(0, C, 1, unroll=2)(row_body)
                pltpu.async_copy(
                    ob.at[b], out_hbm.at[pl.ds(wbase + ck * C, C)], semo[b])

                nk = ck + NBUF

                @pl.when(nk < CHUNKS)
                def _():
                    issue(nk, b)
            return carry

        lax.fori_loop(0, CHUNKS // NBUF, body, 0)
        for b in range(NBUF):
            pltpu.make_async_copy(
                ob.at[b], out_hbm.at[pl.ds(0, C)], semo[b]).wait()

    return k(src3, E0, E1, E2, P0f, P1f, P2f)


def kernel(src, E0, E1, E2, P0, P1, P2):
    src3 = src.reshape(NW, CHUNKS, C)
    P0f = P0.reshape(-1, D)
    P1f = P1.reshape(-1, D)
    P2f = P2.reshape(-1, D)
    out = _matryoshka_sc(src3, E0, E1, E2, P0f, P1f, P2f)
    return out.reshape(B, S, D)


# speedup vs baseline: 9.8356x; 1.0034x over previous
import functools

import jax
import jax.numpy as jnp
from jax import lax
from jax.experimental import pallas as pl
from jax.experimental.pallas import tpu as pltpu
from jax.experimental.pallas import tpu_sc as plsc

B, S, D, V = 1024, 200, 128, 100000
NC, NS = 2, 16
NW = NC * NS
NPW = (B * S) // NW
C = 40
CHUNKS = NPW // C
PC = S // C
LANES = 16
CD = D // LANES
NBUF = 2


def _matryoshka_sc(src3, E0, E1, E2, P0f, P1f, P2f):
    mesh = plsc.VectorSubcoreMesh(core_axis_name="c", subcore_axis_name="s")

    @functools.partial(
        pl.kernel,
        mesh=mesh,
        out_type=jax.ShapeDtypeStruct((B * S, D), jnp.float32),
        scratch_types=[
            pltpu.VMEM((CHUNKS, C), jnp.int32),
            pltpu.VMEM((NBUF, C, D), jnp.float32),
            pltpu.VMEM((NBUF, C, D), jnp.float32),
            pltpu.VMEM((NBUF, C, D), jnp.float32),
            pltpu.VMEM((NBUF, C, D), jnp.float32),
            pltpu.VMEM((S, D), jnp.float32),
            pltpu.VMEM((S, D), jnp.float32),
            pltpu.SemaphoreType.DMA,
            pltpu.SemaphoreType.DMA,
            pltpu.SemaphoreType.DMA,
            pltpu.SemaphoreType.DMA,
        ],
    )
    def k(src_hbm, e0, e1, e2, p0, p1, p2, out_hbm,
          idx_all, g0, g1, g2, ob, ppos, ptmp,
          semg0, semg1, semo0, semo1):
        semg = (semg0, semg1)
        semo = (semo0, semo1)
        wid = lax.axis_index("s") * NC + lax.axis_index("c")
        wbase = wid * NPW

        pltpu.sync_copy(src_hbm.at[wid], idx_all)

        def issue(ck, b):
            idx = idx_all.at[ck]
            pltpu.async_copy(e0.at[idx], g0.at[b], semg[b])
            pltpu.async_copy(e1.at[idx], g1.at[b], semg[b])
            pltpu.async_copy(e2.at[idx], g2.at[b], semg[b])

        def wait_gathers(b):
            for gbuf in (g0, g1, g2):
                pltpu.make_async_copy(
                    e0.at[pl.ds(0, C)], gbuf.at[b], semg[b]).wait()

        issue(0, 0)
        issue(1, 1)

        pltpu.sync_copy(p0.at[pl.ds(0, S)], ppos)
        pltpu.sync_copy(p1.at[pl.ds(0, S)], ptmp)

        def _acc_row(r):
            for c in range(CD):
                sl = pl.ds(c * LANES, LANES)
                ppos[r, sl] = ppos[r, sl] + ptmp[r, sl]

        plsc.parallel_loop(0, S, 1, unroll=2)(_acc_row)
        pltpu.sync_copy(p2.at[pl.ds(0, S)], ptmp)
        plsc.parallel_loop(0, S, 1, unroll=2)(_acc_row)

        def body(i, carry):
            for b in range(NBUF):
                ck = i * NBUF + b
                wait_gathers(b)

                @pl.when(i > 0)
                def _():
                    pltpu.make_async_copy(
                        ob.at[b], out_hbm.at[pl.ds(0, C)], semo[b]).wait()

                pbase = (ck % PC) * C

                def row_body(r):
                    pr = pbase + r
                    for c in range(CD):
                        sl = pl.ds(c * LANES, LANES)
                        ob[b, r, sl] = (g0[b, r, sl] + g1[b, r, sl]
                                        + g2[b, r, sl] + ppos[pr, sl])

                plsc.parallel_loop(0, C, 1, unroll=4)(row_body)
                pltpu.async_copy(
                    ob.at[b], out_hbm.at[pl.ds(wbase + ck * C, C)], semo[b])

                nk = ck + NBUF

                @pl.when(nk < CHUNKS)
                def _():
                    issue(nk, b)
            return carry

        lax.fori_loop(0, CHUNKS // NBUF, body, 0)
        for b in range(NBUF):
            pltpu.make_async_copy(
                ob.at[b], out_hbm.at[pl.ds(0, C)], semo[b]).wait()

    return k(src3, E0, E1, E2, P0f, P1f, P2f)


def kernel(src, E0, E1, E2, P0, P1, P2):
    src3 = src.reshape(NW, CHUNKS, C)
    P0f = P0.reshape(-1, D)
    P1f = P1.reshape(-1, D)
    P2f = P2.reshape(-1, D)
    out = _matryoshka_sc(src3, E0, E1, E2, P0f, P1f, P2f)
    return out.reshape(B, S, D)


# speedup vs baseline: 10.9205x; 1.1103x over previous
import functools

import jax
import jax.numpy as jnp
from jax import lax
from jax.experimental import pallas as pl
from jax.experimental.pallas import tpu as pltpu
from jax.experimental.pallas import tpu_sc as plsc

B, S, D, V = 1024, 200, 128, 100000
NC, NS = 2, 16
NW = NC * NS
BW = B // NW
PQ = 2
C = PQ * BW
CHUNKS = S // PQ
LANES = 16
CD = D // LANES
NBUF = 2


def _matryoshka_sc(src4, oidx4, E0, E1, E2, P0f, P1f, P2f):
    mesh = plsc.VectorSubcoreMesh(core_axis_name="c", subcore_axis_name="s")

    @functools.partial(
        pl.kernel,
        mesh=mesh,
        out_type=jax.ShapeDtypeStruct((B * S, D), jnp.float32),
        scratch_types=[
            pltpu.VMEM((CHUNKS, C), jnp.int32),
            pltpu.VMEM((CHUNKS, C), jnp.int32),
            pltpu.VMEM((NBUF, C, D), jnp.float32),
            pltpu.VMEM((NBUF, C, D), jnp.float32),
            pltpu.VMEM((NBUF, C, D), jnp.float32),
            pltpu.VMEM((NBUF, C, D), jnp.float32),
            pltpu.VMEM((S, D), jnp.float32),
            pltpu.SemaphoreType.DMA,
            pltpu.SemaphoreType.DMA,
            pltpu.SemaphoreType.DMA,
            pltpu.SemaphoreType.DMA,
        ],
    )
    def k(src_hbm, oidx_hbm, e0, e1, e2, p0, p1, p2, out_hbm,
          idx_all, oidx_all, g0, g1, g2, ob, ppos,
          semg0, semg1, semo0, semo1):
        semg = (semg0, semg1)
        semo = (semo0, semo1)
        wid = lax.axis_index("s") * NC + lax.axis_index("c")

        pltpu.sync_copy(src_hbm.at[wid], idx_all)
        pltpu.sync_copy(oidx_hbm.at[wid], oidx_all)

        def issue(ck, b):
            idx = idx_all.at[ck]
            pltpu.async_copy(e0.at[idx], g0.at[b], semg[b])
            pltpu.async_copy(e1.at[idx], g1.at[b], semg[b])
            pltpu.async_copy(e2.at[idx], g2.at[b], semg[b])

        def wait_gathers(b):
            for gbuf in (g0, g1, g2):
                pltpu.make_async_copy(
                    e0.at[pl.ds(0, C)], gbuf.at[b], semg[b]).wait()

        issue(0, 0)
        issue(1, 1)

        pltpu.sync_copy(p0.at[pl.ds(0, S)], ppos)
        for ptab in (p1, p2):
            for r0, n in ((0, C), (C, C), (2 * C, C), (3 * C, S - 3 * C)):
                tmp = ob.at[0]
                pltpu.sync_copy(ptab.at[pl.ds(r0, n)], tmp.at[pl.ds(0, n)])

                def _acc_row(r):
                    for c in range(CD):
                        sl = pl.ds(c * LANES, LANES)
                        ppos[r0 + r, sl] = ppos[r0 + r, sl] + tmp[r, sl]

                plsc.parallel_loop(0, n, 1, unroll=2)(_acc_row)

        def body(i, carry):
            for b in range(NBUF):
                ck = i * NBUF + b
                wait_gathers(b)

                @pl.when(i > 0)
                def _():
                    pltpu.make_async_copy(
                        ob.at[b], out_hbm.at[oidx_all.at[0]], semo[b]).wait()

                for q in range(PQ):
                    pr = ck * PQ + q
                    pos = [ppos[pr, pl.ds(c * LANES, LANES)]
                           for c in range(CD)]

                    def row_body(r, _pos=pos):
                        for c in range(CD):
                            sl = pl.ds(c * LANES, LANES)
                            ob[b, r, sl] = (g0[b, r, sl] + g1[b, r, sl]
                                            + g2[b, r, sl] + _pos[c])

                    plsc.parallel_loop(q * BW, (q + 1) * BW, 1,
                                       unroll=4)(row_body)

                pltpu.async_copy(
                    ob.at[b], out_hbm.at[oidx_all.at[ck]], semo[b])

                nk = ck + NBUF

                @pl.when(nk < CHUNKS)
                def _():
                    issue(nk, b)
            return carry

        lax.fori_loop(0, CHUNKS // NBUF, body, 0)
        for b in range(NBUF):
            pltpu.make_async_copy(
                ob.at[b], out_hbm.at[oidx_all.at[0]], semo[b]).wait()

    return k(src4, oidx4, E0, E1, E2, P0f, P1f, P2f)


def kernel(src, E0, E1, E2, P0, P1, P2):
    src4 = src.reshape(NW, BW, CHUNKS, PQ).transpose(0, 2, 3, 1)
    src4 = src4.reshape(NW, CHUNKS, C)
    brow = (jnp.arange(NW)[:, None, None, None] * BW
            + jnp.arange(BW)[None, None, None, :])
    spos = (jnp.arange(CHUNKS)[None, :, None, None] * PQ
            + jnp.arange(PQ)[None, None, :, None])
    oidx4 = (brow * S + spos).astype(jnp.int32).reshape(NW, CHUNKS, C)
    P0f = P0.reshape(-1, D)
    P1f = P1.reshape(-1, D)
    P2f = P2.reshape(-1, D)
    out = _matryoshka_sc(src4, oidx4, E0, E1, E2, P0f, P1f, P2f)
    return out.reshape(B, S, D)


# speedup vs baseline: 11.7128x; 1.0726x over previous
import functools

import jax
import jax.numpy as jnp
from jax import lax
from jax.experimental import pallas as pl
from jax.experimental.pallas import tpu as pltpu
from jax.experimental.pallas import tpu_sc as plsc

B, S, D, V = 1024, 200, 128, 100000
NC, NS = 2, 16
NW = NC * NS
BW = B // NW
PQ = 2
C = PQ * BW
CHUNKS = S // PQ
LANES = 16
CD = D // LANES
NBUF = 4


def _matryoshka_sc(src4, oidx4, E0, E1, E2, P0f, P1f, P2f):
    mesh = plsc.VectorSubcoreMesh(core_axis_name="c", subcore_axis_name="s")

    @functools.partial(
        pl.kernel,
        mesh=mesh,
        out_type=jax.ShapeDtypeStruct((B * S, D), jnp.float32),
        scratch_types=[
            pltpu.VMEM((CHUNKS, C), jnp.int32),
            pltpu.VMEM((CHUNKS, C), jnp.int32),
            pltpu.VMEM((NBUF, C, D), jnp.float32),
            pltpu.VMEM((NBUF, C, D), jnp.float32),
            pltpu.VMEM((NBUF, C, D), jnp.float32),
            pltpu.VMEM((NBUF, 3, PQ, D), jnp.float32),
            pltpu.SemaphoreType.DMA,
            pltpu.SemaphoreType.DMA,
            pltpu.SemaphoreType.DMA,
            pltpu.SemaphoreType.DMA,
            pltpu.SemaphoreType.DMA,
            pltpu.SemaphoreType.DMA,
            pltpu.SemaphoreType.DMA,
            pltpu.SemaphoreType.DMA,
        ],
    )
    def k(src_hbm, oidx_hbm, e0, e1, e2, p0, p1, p2, out_hbm,
          idx_all, oidx_all, acc, g1, g2, pbuf,
          sg0, sg1, sg2, sg3, so0, so1, so2, so3):
        semg = (sg0, sg1, sg2, sg3)
        semo = (so0, so1, so2, so3)
        wid = lax.axis_index("s") * NC + lax.axis_index("c")

        pltpu.sync_copy(src_hbm.at[wid], idx_all)
        pltpu.sync_copy(oidx_hbm.at[wid], oidx_all)

        def issue_far(ck, b):
            idx = idx_all.at[ck]
            pltpu.async_copy(e1.at[idx], g1.at[b], semg[b])
            pltpu.async_copy(e2.at[idx], g2.at[b], semg[b])
            for t, ptab in enumerate((p0, p1, p2)):
                pltpu.async_copy(ptab.at[pl.ds(ck * PQ, PQ)],
                                 pbuf.at[b, t], semg[b])

        def issue_e0(ck, b):
            pltpu.async_copy(e0.at[idx_all.at[ck]], acc.at[b], semg[b])

        def wait_chunk(b):
            pltpu.make_async_copy(e0.at[pl.ds(0, C)], acc.at[b],
                                  semg[b]).wait()
            pltpu.make_async_copy(e0.at[pl.ds(0, C)], g1.at[b],
                                  semg[b]).wait()
            pltpu.make_async_copy(e0.at[pl.ds(0, C)], g2.at[b],
                                  semg[b]).wait()
            for t in range(3):
                pltpu.make_async_copy(p0.at[pl.ds(0, PQ)], pbuf.at[b, t],
                                      semg[b]).wait()

        def wait_store(b):
            pltpu.make_async_copy(acc.at[b], out_hbm.at[oidx_all.at[0]],
                                  semo[b]).wait()

        for ck in range(NBUF):
            issue_far(ck, ck)
        for ck in range(2):
            issue_e0(ck, ck)

        def step(ck, b):
            eb = (b + 2) % NBUF

            @pl.when(ck >= 2)
            def _():
                wait_store(eb)

            @pl.when(ck + 2 < CHUNKS)
            def _():
                issue_e0(ck + 2, eb)

            wait_chunk(b)

            for q in range(PQ):
                pos = []
                for c in range(CD):
                    sl = pl.ds(c * LANES, LANES)
                    pos.append(pbuf[b, 0, q, sl] + pbuf[b, 1, q, sl]
                               + pbuf[b, 2, q, sl])

                def row_body(r, _pos=pos):
                    for c in range(CD):
                        sl = pl.ds(c * LANES, LANES)
                        plsc.addupdate(acc.at[b, r, sl],
                                       g1[b, r, sl] + g2[b, r, sl] + _pos[c])

                plsc.parallel_loop(q * BW, (q + 1) * BW, 1,
                                   unroll=4)(row_body)

            pltpu.async_copy(acc.at[b], out_hbm.at[oidx_all.at[ck]], semo[b])

            @pl.when(ck + NBUF < CHUNKS)
            def _():
                issue_far(ck + NBUF, b)

        def body(i, carry):
            for b in range(NBUF):
                step(i * NBUF + b, b)
            return carry

        lax.fori_loop(0, CHUNKS // NBUF, body, 0)
        wait_store(2)
        wait_store(3)

    return k(src4, oidx4, E0, E1, E2, P0f, P1f, P2f)


def kernel(src, E0, E1, E2, P0, P1, P2):
    src4 = src.reshape(NW, BW, CHUNKS, PQ).transpose(0, 2, 3, 1)
    src4 = src4.reshape(NW, CHUNKS, C)
    brow = (jnp.arange(NW)[:, None, None, None] * BW
            + jnp.arange(BW)[None, None, None, :])
    spos = (jnp.arange(CHUNKS)[None, :, None, None] * PQ
            + jnp.arange(PQ)[None, None, :, None])
    oidx4 = (brow * S + spos).astype(jnp.int32).reshape(NW, CHUNKS, C)
    P0f = P0.reshape(-1, D)
    P1f = P1.reshape(-1, D)
    P2f = P2.reshape(-1, D)
    out = _matryoshka_sc(src4, oidx4, E0, E1, E2, P0f, P1f, P2f)
    return out.reshape(B, S, D)
